# Initial kernel scaffold; baseline (speedup 1.0000x reference)
import jax
import jax.numpy as jnp
from jax.experimental import pallas as pl


def kernel(x, edge_index, W1, b1, W2, b2):
    raise NotImplementedError("write your pallas kernel here")


# speedup vs baseline: 13.6437x; 13.6437x over previous
import functools

import jax
import jax.numpy as jnp
from jax import lax
from jax.experimental import pallas as pl
from jax.experimental.pallas import tpu as pltpu
from jax.experimental.pallas import tpu_sc as plsc

NC = 2
NS = 16
K = 128
BLK = 512


def _sc_degree(dst_pad, n_acc, e_pad):
    nw = NC * NS
    ew = e_pad // nw
    nb = ew // K
    rows_pt = n_acc // NS
    mesh = plsc.VectorSubcoreMesh(core_axis_name="c", subcore_axis_name="s")

    @functools.partial(
        pl.kernel,
        out_type=jax.ShapeDtypeStruct((NC, n_acc, 128), jnp.float32),
        mesh=mesh,
        scratch_types=[
            pltpu.VMEM((K,), jnp.int32),
            pltpu.VMEM((K, 128), jnp.float32),
            pltpu.VMEM((K, 128), jnp.float32),
            pltpu.VMEM_SHARED((n_acc, 128), jnp.float32),
        ],
    )
    def kern(dst_hbm, out_hbm, didx, ones_v, zer_v, acc):
        c = lax.axis_index("c")
        s = lax.axis_index("s")
        wid = c * NS + s

        def fill(i, carry):
            for kk in range(8):
                ones_v[i, pl.ds(kk * 16, 16)] = jnp.full((16,), 1.0, jnp.float32)
                zer_v[i, pl.ds(kk * 16, 16)] = jnp.zeros((16,), jnp.float32)
            return carry

        lax.fori_loop(0, K, fill, 0)

        t0 = s * rows_pt

        def zacc(j, carry):
            pltpu.sync_copy(zer_v, acc.at[pl.ds(t0 + j * K, K)])
            return carry

        lax.fori_loop(0, rows_pt // K, zacc, 0)
        plsc.subcore_barrier()

        wbase = wid * ew

        def body(i, carry):
            pltpu.sync_copy(dst_hbm.at[pl.ds(wbase + i * K, K)], didx)
            pltpu.sync_copy(ones_v, acc.at[didx], add=True)
            return carry

        lax.fori_loop(0, nb, body, 0)
        plsc.subcore_barrier()
        pltpu.sync_copy(acc.at[pl.ds(t0, rows_pt)],
                        out_hbm.at[c, pl.ds(t0, rows_pt)])

    return kern(dst_pad)


def _sc_aggregate(xs_pad, src_pad, dst_pad, n_acc, e_pad):
    nw = NC * NS
    ew = e_pad // nw
    nb = ew // K
    rows_pt = n_acc // NS
    mesh = plsc.VectorSubcoreMesh(core_axis_name="c", subcore_axis_name="s")

    @functools.partial(
        pl.kernel,
        out_type=jax.ShapeDtypeStruct((NC, n_acc, 128), jnp.float32),
        mesh=mesh,
        scratch_types=[
            pltpu.VMEM((K,), jnp.int32),
            pltpu.VMEM((K,), jnp.int32),
            pltpu.VMEM((K, 128), jnp.float32),
            pltpu.VMEM_SHARED((n_acc, 128), jnp.float32),
            pltpu.SemaphoreType.DMA,
        ],
    )
    def kern(xs_hbm, src_hbm, dst_hbm, out_hbm, sidx, didx, rows, acc, gsem):
        c = lax.axis_index("c")
        s = lax.axis_index("s")
        wid = c * NS + s

        def zrow(i, carry):
            for kk in range(8):
                rows[i, pl.ds(kk * 16, 16)] = jnp.zeros((16,), jnp.float32)
            return carry

        lax.fori_loop(0, K, zrow, 0)

        t0 = s * rows_pt

        def zacc(j, carry):
            pltpu.sync_copy(rows, acc.at[pl.ds(t0 + j * K, K)])
            return carry

        lax.fori_loop(0, rows_pt // K, zacc, 0)
        plsc.subcore_barrier()

        wbase = wid * ew

        def body(i, carry):
            base = wbase + i * K
            pltpu.sync_copy(src_hbm.at[pl.ds(base, K)], sidx)
            pltpu.async_copy(xs_hbm.at[sidx], rows, gsem).wait()
            pltpu.sync_copy(dst_hbm.at[pl.ds(base, K)], didx)
            pltpu.sync_copy(rows, acc.at[didx], add=True)
            return carry

        lax.fori_loop(0, nb, body, 0)
        plsc.subcore_barrier()
        pltpu.sync_copy(acc.at[pl.ds(t0, rows_pt)],
                        out_hbm.at[c, pl.ds(t0, rows_pt)])

    return kern(xs_pad, src_pad, dst_pad)


def _dinv_block(dref):
    deg = dref[0, :, 0:1] + dref[1, :, 0:1] + 1.0
    return lax.rsqrt(deg)


def _tc_prescale(degp, x_pad, n_acc):
    def body(dref, xref, oref):
        oref[...] = xref[...] * _dinv_block(dref)

    return pl.pallas_call(
        body,
        grid=(n_acc // BLK,),
        in_specs=[
            pl.BlockSpec((2, BLK, 128), lambda i: (0, i, 0)),
            pl.BlockSpec((BLK, 128), lambda i: (i, 0)),
        ],
        out_specs=pl.BlockSpec((BLK, 128), lambda i: (i, 0)),
        out_shape=jax.ShapeDtypeStruct((n_acc, 128), jnp.float32),
    )(degp, x_pad)


def _tc_mid(aggp, degp, xs, W1, b1, W2, n_acc, hid):
    def body(aref, dref, xref, w1, b1r, w2, oref):
        dinv = _dinv_block(dref)
        agg = (aref[0] + aref[1] + xref[...]) * dinv
        h = jnp.dot(agg, w1[...], preferred_element_type=jnp.float32)
        h = jnp.maximum(h + b1r[...], 0.0)
        z = jnp.dot(h, w2[...], preferred_element_type=jnp.float32)
        oref[...] = z * dinv

    return pl.pallas_call(
        body,
        grid=(n_acc // BLK,),
        in_specs=[
            pl.BlockSpec((2, BLK, 128), lambda i: (0, i, 0)),
            pl.BlockSpec((2, BLK, 128), lambda i: (0, i, 0)),
            pl.BlockSpec((BLK, 128), lambda i: (i, 0)),
            pl.BlockSpec((128, hid), lambda i: (0, 0)),
            pl.BlockSpec((1, hid), lambda i: (0, 0)),
            pl.BlockSpec((hid, 128), lambda i: (0, 0)),
        ],
        out_specs=pl.BlockSpec((BLK, 128), lambda i: (i, 0)),
        out_shape=jax.ShapeDtypeStruct((n_acc, 128), jnp.float32),
    )(aggp, degp, xs, W1, b1, W2)


def _tc_final(aggp, degp, zs, b2, n_acc):
    def body(aref, dref, zref, b2r, oref):
        dinv = _dinv_block(dref)
        v = (aref[0] + aref[1] + zref[...]) * dinv + b2r[...]
        oref[...] = jnp.maximum(v, 0.0)

    return pl.pallas_call(
        body,
        grid=(n_acc // BLK,),
        in_specs=[
            pl.BlockSpec((2, BLK, 128), lambda i: (0, i, 0)),
            pl.BlockSpec((2, BLK, 128), lambda i: (0, i, 0)),
            pl.BlockSpec((BLK, 128), lambda i: (i, 0)),
            pl.BlockSpec((1, 128), lambda i: (0, 0)),
        ],
        out_specs=pl.BlockSpec((BLK, 128), lambda i: (i, 0)),
        out_shape=jax.ShapeDtypeStruct((n_acc, 128), jnp.float32),
    )(aggp, degp, zs, b2)


def kernel(x, edge_index, W1, b1, W2, b2):
    n, _ = x.shape
    e = edge_index.shape[1]
    hid = W1.shape[1]

    nw = NC * NS
    chunk = nw * K
    e_pad = ((e + chunk - 1) // chunk) * chunk
    stride = NS * K
    n_acc = ((n + 1 + stride - 1) // stride) * stride

    src = edge_index[0].astype(jnp.int32)
    dst = edge_index[1].astype(jnp.int32)
    pad = e_pad - e
    src_p = jnp.concatenate([src, jnp.zeros((pad,), jnp.int32)])
    dst_p = jnp.concatenate([dst, jnp.full((pad,), n, jnp.int32)])
    x_p = jnp.pad(x, ((0, n_acc - n), (0, 0)))
    b1r = b1.reshape(1, hid)
    b2r = b2.reshape(1, -1)

    degp = _sc_degree(dst_p, n_acc, e_pad)
    xs = _tc_prescale(degp, x_p, n_acc)
    aggp1 = _sc_aggregate(xs, src_p, dst_p, n_acc, e_pad)
    zs = _tc_mid(aggp1, degp, xs, W1, b1r, W2, n_acc, hid)
    aggp2 = _sc_aggregate(zs, src_p, dst_p, n_acc, e_pad)
    out = _tc_final(aggp2, degp, zs, b2r, n_acc)
    return out[:n]
